# Initial kernel scaffold; baseline (speedup 1.0000x reference)
import jax
import jax.numpy as jnp
from jax.experimental import pallas as pl


def kernel(nodes, edges, receivers, senders, global_latent, node_graph_idx, edge_graph_idx, W_e, b_e, W_n, b_n, W_g, b_g, rn_w):
    raise NotImplementedError("write your pallas kernel here")


# speedup vs baseline: 9.3106x; 9.3106x over previous
import functools

import jax
import jax.numpy as jnp
from jax import lax
from jax.experimental import pallas as pl
from jax.experimental.pallas import tpu as pltpu
from jax.experimental.pallas import tpu_sc as plsc

N = 10000
E = 320000
G = 16
D = 128

NC = 2
NS = 16
NW = NC * NS
ROWS_W = E // NW
CH = 80
NCHUNK = ROWS_W // CH

BLK = 2560
NSTEP = E // BLK

_HP = lax.Precision.HIGHEST
_F32 = jnp.float32


def _sc_mesh():
    return plsc.VectorSubcoreMesh(core_axis_name="c", subcore_axis_name="s")


def _prep_body(nodes_ref, glob_ref, we_ref, be_ref, ps_ref, pr_ref, gpe_ref):
    nodes = nodes_ref[...]
    ps_ref[...] = jnp.dot(nodes, we_ref[D:2 * D, :], precision=_HP)
    pr_ref[...] = jnp.dot(nodes, we_ref[2 * D:3 * D, :], precision=_HP)
    gpe_ref[...] = jnp.dot(glob_ref[...], we_ref[3 * D:, :], precision=_HP) + be_ref[...]


def _prep_tc(nodes, glob, W_e, b_e2):
    return pl.pallas_call(
        _prep_body,
        out_shape=(
            jax.ShapeDtypeStruct((N, D), _F32),
            jax.ShapeDtypeStruct((N, D), _F32),
            jax.ShapeDtypeStruct((G, D), _F32),
        ),
    )(nodes, glob, W_e, b_e2)


@functools.cache
def _make_gather_sc():
    @functools.partial(
        pl.kernel,
        out_type=(
            jax.ShapeDtypeStruct((E, D), _F32),
            jax.ShapeDtypeStruct((E, D), _F32),
        ),
        mesh=_sc_mesh(),
        scratch_types=[
            pltpu.VMEM((CH,), jnp.int32),
            pltpu.VMEM((CH,), jnp.int32),
            pltpu.VMEM((CH, D), _F32),
            pltpu.VMEM((CH, D), _F32),
            pltpu.SemaphoreType.DMA,
            pltpu.SemaphoreType.DMA,
        ],
    )
    def _gather_sc(ps, pr, snd, rcv, gs, gr, sidx, ridx, srow, rrow, sem1, sem2):
        wid = lax.axis_index("s") * NC + lax.axis_index("c")
        base = wid * ROWS_W

        def chunk(i, carry):
            off = base + i * CH
            pltpu.sync_copy(snd.at[pl.ds(off, CH)], sidx)
            pltpu.sync_copy(rcv.at[pl.ds(off, CH)], ridx)
            c1 = pltpu.async_copy(ps.at[sidx], srow, sem1)
            c2 = pltpu.async_copy(pr.at[ridx], rrow, sem2)
            c1.wait()
            c2.wait()
            pltpu.sync_copy(srow, gs.at[pl.ds(off, CH)])
            pltpu.sync_copy(rrow, gr.at[pl.ds(off, CH)])
            return carry

        lax.fori_loop(0, NCHUNK, chunk, 0)

    return _gather_sc


def _edge_body(mult_ref, edges_ref, gs_ref, gr_ref, eg_ref, w1_ref, gpe_ref,
               eu_ref, eo_ref, eagg_ref):
    step = pl.program_id(0)
    ohT = (lax.broadcasted_iota(jnp.int32, (G, BLK), 0)
           == eg_ref[0, :, :]).astype(_F32)
    edges = edges_ref[...]
    pre = jnp.dot(edges, w1_ref[...], precision=_HP)
    pre += gs_ref[...] + gr_ref[...]
    pre += lax.dot_general(ohT, gpe_ref[...], (((0,), (0,)), ((), ())),
                           precision=_HP)
    eu = jnp.maximum(pre, 0.0)
    eu_ref[...] = eu
    eo_ref[...] = edges + mult_ref[0, 0] * eu

    @pl.when(step == 0)
    def _():
        eagg_ref[...] = jnp.zeros_like(eagg_ref)

    eagg_ref[...] += lax.dot_general(ohT, eu, (((1,), (0,)), ((), ())),
                                     precision=_HP)


def _edge_tc(multa, edges, gs, gr, eg3, W1, gpe):
    return pl.pallas_call(
        _edge_body,
        grid=(NSTEP,),
        in_specs=[
            pl.BlockSpec((1, 1), lambda i: (0, 0)),
            pl.BlockSpec((BLK, D), lambda i: (i, 0)),
            pl.BlockSpec((BLK, D), lambda i: (i, 0)),
            pl.BlockSpec((BLK, D), lambda i: (i, 0)),
            pl.BlockSpec((1, 1, BLK), lambda i: (i, 0, 0)),
            pl.BlockSpec((D, D), lambda i: (0, 0)),
            pl.BlockSpec((G, D), lambda i: (0, 0)),
        ],
        out_specs=[
            pl.BlockSpec((BLK, D), lambda i: (i, 0)),
            pl.BlockSpec((BLK, D), lambda i: (i, 0)),
            pl.BlockSpec((G, D), lambda i: (0, 0)),
        ],
        out_shape=(
            jax.ShapeDtypeStruct((E, D), _F32),
            jax.ShapeDtypeStruct((E, D), _F32),
            jax.ShapeDtypeStruct((G, D), _F32),
        ),
    )(multa, edges, gs, gr, eg3, W1, gpe)


@functools.cache
def _make_scatter_sc():
    @functools.partial(
        pl.kernel,
        out_type=jax.ShapeDtypeStruct((NC, N, D), _F32),
        mesh=_sc_mesh(),
        scratch_types=[
            pltpu.VMEM((CH,), jnp.int32),
            pltpu.VMEM((CH, D), _F32),
            pltpu.VMEM_SHARED((N, D), _F32),
        ],
    )
    def _scatter_sc(eu, rcv, zer, adj, ridx, row, accum):
        cid = lax.axis_index("c")
        sid = lax.axis_index("s")

        @pl.when(sid == 0)
        def _():
            pltpu.sync_copy(zer, accum)

        plsc.subcore_barrier()
        base = (sid * NC + cid) * ROWS_W

        def chunk(i, carry):
            off = base + i * CH
            pltpu.sync_copy(rcv.at[pl.ds(off, CH)], ridx)
            pltpu.sync_copy(eu.at[pl.ds(off, CH)], row)
            pltpu.sync_copy(row, accum.at[ridx], add=True)
            return carry

        lax.fori_loop(0, NCHUNK, chunk, 0)
        plsc.subcore_barrier()

        @pl.when(sid == 0)
        def _():
            pltpu.sync_copy(accum, adj.at[cid])

    return _scatter_sc


NBLK = 2000
NNSTEP = N // NBLK


def _node_body(mult_ref, nodes_ref, adj_ref, ng_ref, glob_ref, wn_ref, bn_ref,
               no_ref, nagg_ref):
    step = pl.program_id(0)
    mult = mult_ref[0, 0]
    nodes = nodes_ref[...]
    adj = adj_ref[0] + adj_ref[1]
    ohT = (lax.broadcasted_iota(jnp.int32, (G, NBLK), 0)
           == ng_ref[0, :, :]).astype(_F32)
    gwn = jnp.dot(glob_ref[...], wn_ref[2 * D:, :], precision=_HP)
    pre = jnp.dot(nodes, wn_ref[:D, :], precision=_HP)
    pre += jnp.dot(adj, wn_ref[D:2 * D, :], precision=_HP)
    pre += lax.dot_general(ohT, gwn, (((0,), (0,)), ((), ())), precision=_HP)
    pre += bn_ref[...]
    nu = jnp.maximum(pre, 0.0)
    no_ref[...] = nodes + mult * nu

    @pl.when(step == 0)
    def _():
        nagg_ref[...] = jnp.zeros_like(nagg_ref)

    nagg_ref[...] += lax.dot_general(ohT, nu, (((1,), (0,)), ((), ())),
                                     precision=_HP)


def _node_tc(multa, nodes, adj, ng3, glob, W_n, b_n2):
    return pl.pallas_call(
        _node_body,
        grid=(NNSTEP,),
        in_specs=[
            pl.BlockSpec((1, 1), lambda i: (0, 0)),
            pl.BlockSpec((NBLK, D), lambda i: (i, 0)),
            pl.BlockSpec((NC, NBLK, D), lambda i: (0, i, 0)),
            pl.BlockSpec((1, 1, NBLK), lambda i: (i, 0, 0)),
            pl.BlockSpec((G, D), lambda i: (0, 0)),
            pl.BlockSpec((3 * D, D), lambda i: (0, 0)),
            pl.BlockSpec((1, D), lambda i: (0, 0)),
        ],
        out_specs=[
            pl.BlockSpec((NBLK, D), lambda i: (i, 0)),
            pl.BlockSpec((G, D), lambda i: (0, 0)),
        ],
        out_shape=(
            jax.ShapeDtypeStruct((N, D), _F32),
            jax.ShapeDtypeStruct((G, D), _F32),
        ),
    )(multa, nodes, adj, ng3, glob, W_n, b_n2)


def _glob_body(mult_ref, nagg_ref, eagg_ref, glob_ref, wg_ref, bg_ref, go_ref):
    glob = glob_ref[...]
    gpre = jnp.dot(nagg_ref[...], wg_ref[:D, :], precision=_HP)
    gpre += jnp.dot(eagg_ref[...], wg_ref[D:2 * D, :], precision=_HP)
    gpre += jnp.dot(glob, wg_ref[2 * D:, :], precision=_HP)
    gpre += bg_ref[...]
    go_ref[...] = glob + mult_ref[0, 0] * jnp.maximum(gpre, 0.0)


def _glob_tc(multa, nagg, eagg, glob, W_g, b_g2):
    return pl.pallas_call(
        _glob_body,
        out_shape=jax.ShapeDtypeStruct((G, D), _F32),
    )(multa, nagg, eagg, glob, W_g, b_g2)


def kernel(nodes, edges, receivers, senders, global_latent, node_graph_idx,
           edge_graph_idx, W_e, b_e, W_n, b_n, W_g, b_g, rn_w):
    nodes2 = nodes[0]
    edges2 = edges[0]
    recv = receivers[0]
    snd = senders[0]
    glob = global_latent[0]
    ng = node_graph_idx[0]
    eg = edge_graph_idx[0]
    multa = jax.nn.softplus(rn_w).reshape(1, 1)

    ps, pr, gpe = _prep_tc(nodes2, glob, W_e, b_e.reshape(1, D))
    gs, gr = _make_gather_sc()(ps, pr, snd, recv)
    eu, eo, eagg = _edge_tc(multa, edges2, gs, gr,
                            eg.reshape(NSTEP, 1, BLK), W_e[:D], gpe)
    adj = _make_scatter_sc()(eu, recv, jnp.zeros((N, D), _F32))
    no, nagg = _node_tc(multa, nodes2, adj, ng.reshape(NNSTEP, 1, NBLK), glob,
                        W_n, b_n.reshape(1, D))
    go = _glob_tc(multa, nagg, eagg, glob, W_g, b_g.reshape(1, D))
    return (no[None], eo[None], go[None])


# speedup vs baseline: 11.9124x; 1.2794x over previous
import functools

import jax
import jax.numpy as jnp
from jax import lax
from jax.experimental import pallas as pl
from jax.experimental.pallas import tpu as pltpu
from jax.experimental.pallas import tpu_sc as plsc

N = 10000
E = 320000
G = 16
D = 128

NC = 2
NS = 16
NW = NC * NS
ROWS_W = E // NW
CH = 40
NCHUNK = ROWS_W // CH

BLK = 2560
NSTEP = E // BLK

_HP = lax.Precision.HIGHEST
_F32 = jnp.float32


def _sc_mesh():
    return plsc.VectorSubcoreMesh(core_axis_name="c", subcore_axis_name="s")


def _prep_body(nodes_ref, glob_ref, we_ref, be_ref, ps_ref, pr_ref, gpe_ref):
    nodes = nodes_ref[...]
    ps_ref[...] = jnp.dot(nodes, we_ref[D:2 * D, :], precision=_HP)
    pr_ref[...] = jnp.dot(nodes, we_ref[2 * D:3 * D, :], precision=_HP)
    gpe_ref[...] = jnp.dot(glob_ref[...], we_ref[3 * D:, :], precision=_HP) + be_ref[...]


def _prep_tc(nodes, glob, W_e, b_e2):
    return pl.pallas_call(
        _prep_body,
        out_shape=(
            jax.ShapeDtypeStruct((N, D), _F32),
            jax.ShapeDtypeStruct((N, D), _F32),
            jax.ShapeDtypeStruct((G, D), _F32),
        ),
    )(nodes, glob, W_e, b_e2)


RB = 5
NJ = NCHUNK // RB
KS = 2
NJS = NCHUNK // KS


@functools.cache
def _make_gather_sc():
    @functools.partial(
        pl.kernel,
        out_type=(
            jax.ShapeDtypeStruct((E, D), _F32),
            jax.ShapeDtypeStruct((E, D), _F32),
        ),
        mesh=_sc_mesh(),
        scratch_types=[
            pltpu.VMEM((NCHUNK, CH), jnp.int32),
            pltpu.VMEM((NCHUNK, CH), jnp.int32),
            pltpu.VMEM((RB, CH, D), _F32),
            pltpu.VMEM((RB, CH, D), _F32),
            pltpu.SemaphoreType.DMA((RB,)),
            pltpu.SemaphoreType.DMA((RB,)),
            pltpu.SemaphoreType.DMA((RB,)),
            pltpu.SemaphoreType.DMA((RB,)),
        ],
    )
    def _gather_sc(ps, pr, snd3, rcv3, gs, gr, sidx, ridx, sbuf, rbuf,
                   gsemS, gsemR, wsemS, wsemR):
        wid = lax.axis_index("s") * NC + lax.axis_index("c")
        base = wid * ROWS_W
        pltpu.sync_copy(snd3.at[wid], sidx)
        pltpu.sync_copy(rcv3.at[wid], ridx)

        def _fire(c, k):
            pltpu.async_copy(ps.at[sidx.at[c]], sbuf.at[k], gsemS.at[k])
            pltpu.async_copy(pr.at[ridx.at[c]], rbuf.at[k], gsemR.at[k])

        def _drain_wb(c, k):
            off = base + c * CH
            pltpu.make_async_copy(ps.at[sidx.at[c]], sbuf.at[k],
                                  gsemS.at[k]).wait()
            pltpu.make_async_copy(pr.at[ridx.at[c]], rbuf.at[k],
                                  gsemR.at[k]).wait()
            pltpu.async_copy(sbuf.at[k], gs.at[pl.ds(off, CH)], wsemS.at[k])
            pltpu.async_copy(rbuf.at[k], gr.at[pl.ds(off, CH)], wsemR.at[k])

        def _wait_wb(c, k):
            off = base + c * CH
            pltpu.make_async_copy(sbuf.at[k], gs.at[pl.ds(off, CH)],
                                  wsemS.at[k]).wait()
            pltpu.make_async_copy(rbuf.at[k], gr.at[pl.ds(off, CH)],
                                  wsemR.at[k]).wait()

        for k in range(RB):
            _fire(k, k)

        def ring(j, carry):
            c0 = j * RB
            for k in range(RB):
                _drain_wb(c0 + k, k)
            for k in range(RB):
                _wait_wb(c0 + k, k)
                _fire(c0 + RB + k, k)
            return carry

        lax.fori_loop(0, NJ - 1, ring, 0)
        c0 = (NJ - 1) * RB
        for k in range(RB):
            _drain_wb(c0 + k, k)
        for k in range(RB):
            _wait_wb(c0 + k, k)

    return _gather_sc


def _edge_body(mult_ref, edges_ref, gs_ref, gr_ref, eg_ref, w1_ref, gpe_ref,
               eu_ref, eo_ref, eagg_ref):
    step = pl.program_id(0)
    ohT = (lax.broadcasted_iota(jnp.int32, (G, BLK), 0)
           == eg_ref[0, :, :]).astype(_F32)
    edges = edges_ref[...]
    pre = jnp.dot(edges, w1_ref[...], precision=_HP)
    pre += gs_ref[...] + gr_ref[...]
    pre += lax.dot_general(ohT, gpe_ref[...], (((0,), (0,)), ((), ())),
                           precision=_HP)
    eu = jnp.maximum(pre, 0.0)
    eu_ref[...] = eu
    eo_ref[...] = edges + mult_ref[0, 0] * eu

    @pl.when(step == 0)
    def _():
        eagg_ref[...] = jnp.zeros_like(eagg_ref)

    eagg_ref[...] += lax.dot_general(ohT, eu, (((1,), (0,)), ((), ())),
                                     precision=_HP)


def _edge_tc(multa, edges, gs, gr, eg3, W1, gpe):
    return pl.pallas_call(
        _edge_body,
        grid=(NSTEP,),
        in_specs=[
            pl.BlockSpec((1, 1), lambda i: (0, 0)),
            pl.BlockSpec((BLK, D), lambda i: (i, 0)),
            pl.BlockSpec((BLK, D), lambda i: (i, 0)),
            pl.BlockSpec((BLK, D), lambda i: (i, 0)),
            pl.BlockSpec((1, 1, BLK), lambda i: (i, 0, 0)),
            pl.BlockSpec((D, D), lambda i: (0, 0)),
            pl.BlockSpec((G, D), lambda i: (0, 0)),
        ],
        out_specs=[
            pl.BlockSpec((BLK, D), lambda i: (i, 0)),
            pl.BlockSpec((BLK, D), lambda i: (i, 0)),
            pl.BlockSpec((G, D), lambda i: (0, 0)),
        ],
        out_shape=(
            jax.ShapeDtypeStruct((E, D), _F32),
            jax.ShapeDtypeStruct((E, D), _F32),
            jax.ShapeDtypeStruct((G, D), _F32),
        ),
    )(multa, edges, gs, gr, eg3, W1, gpe)


@functools.cache
def _make_scatter_sc():
    @functools.partial(
        pl.kernel,
        out_type=jax.ShapeDtypeStruct((NC, N, D), _F32),
        mesh=_sc_mesh(),
        scratch_types=[
            pltpu.VMEM((NCHUNK, CH), jnp.int32),
            pltpu.VMEM((KS, CH, D), _F32),
            pltpu.VMEM_SHARED((N, D), _F32),
            pltpu.SemaphoreType.DMA((KS,)),
        ],
    )
    def _scatter_sc(eu, rcv3, zer, adj, ridx, rows, accum, lsem):
        cid = lax.axis_index("c")
        sid = lax.axis_index("s")

        @pl.when(sid == 0)
        def _():
            pltpu.sync_copy(zer, accum)

        wid = sid * NC + cid
        base = wid * ROWS_W
        pltpu.sync_copy(rcv3.at[wid], ridx)
        plsc.subcore_barrier()

        def _fire(c, k):
            off = base + c * CH
            pltpu.async_copy(eu.at[pl.ds(off, CH)], rows.at[k], lsem.at[k])

        def _scat(c, k):
            off = base + c * CH
            pltpu.make_async_copy(eu.at[pl.ds(off, CH)], rows.at[k],
                                  lsem.at[k]).wait()
            pltpu.sync_copy(rows.at[k], accum.at[ridx.at[c]], add=True)

        for k in range(KS):
            _fire(k, k)

        def ring(j, carry):
            c0 = j * KS
            for k in range(KS):
                _scat(c0 + k, k)
                _fire(c0 + KS + k, k)
            return carry

        lax.fori_loop(0, NJS - 1, ring, 0)
        c0 = (NJS - 1) * KS
        for k in range(KS):
            _scat(c0 + k, k)
        plsc.subcore_barrier()

        @pl.when(sid == 0)
        def _():
            pltpu.sync_copy(accum, adj.at[cid])

    return _scatter_sc


NBLK = 2000
NNSTEP = N // NBLK


def _node_body(mult_ref, nodes_ref, adj_ref, ng_ref, glob_ref, wn_ref, bn_ref,
               no_ref, nagg_ref):
    step = pl.program_id(0)
    mult = mult_ref[0, 0]
    nodes = nodes_ref[...]
    adj = adj_ref[0] + adj_ref[1]
    ohT = (lax.broadcasted_iota(jnp.int32, (G, NBLK), 0)
           == ng_ref[0, :, :]).astype(_F32)
    gwn = jnp.dot(glob_ref[...], wn_ref[2 * D:, :], precision=_HP)
    pre = jnp.dot(nodes, wn_ref[:D, :], precision=_HP)
    pre += jnp.dot(adj, wn_ref[D:2 * D, :], precision=_HP)
    pre += lax.dot_general(ohT, gwn, (((0,), (0,)), ((), ())), precision=_HP)
    pre += bn_ref[...]
    nu = jnp.maximum(pre, 0.0)
    no_ref[...] = nodes + mult * nu

    @pl.when(step == 0)
    def _():
        nagg_ref[...] = jnp.zeros_like(nagg_ref)

    nagg_ref[...] += lax.dot_general(ohT, nu, (((1,), (0,)), ((), ())),
                                     precision=_HP)


def _node_tc(multa, nodes, adj, ng3, glob, W_n, b_n2):
    return pl.pallas_call(
        _node_body,
        grid=(NNSTEP,),
        in_specs=[
            pl.BlockSpec((1, 1), lambda i: (0, 0)),
            pl.BlockSpec((NBLK, D), lambda i: (i, 0)),
            pl.BlockSpec((NC, NBLK, D), lambda i: (0, i, 0)),
            pl.BlockSpec((1, 1, NBLK), lambda i: (i, 0, 0)),
            pl.BlockSpec((G, D), lambda i: (0, 0)),
            pl.BlockSpec((3 * D, D), lambda i: (0, 0)),
            pl.BlockSpec((1, D), lambda i: (0, 0)),
        ],
        out_specs=[
            pl.BlockSpec((NBLK, D), lambda i: (i, 0)),
            pl.BlockSpec((G, D), lambda i: (0, 0)),
        ],
        out_shape=(
            jax.ShapeDtypeStruct((N, D), _F32),
            jax.ShapeDtypeStruct((G, D), _F32),
        ),
    )(multa, nodes, adj, ng3, glob, W_n, b_n2)


def _glob_body(mult_ref, nagg_ref, eagg_ref, glob_ref, wg_ref, bg_ref, go_ref):
    glob = glob_ref[...]
    gpre = jnp.dot(nagg_ref[...], wg_ref[:D, :], precision=_HP)
    gpre += jnp.dot(eagg_ref[...], wg_ref[D:2 * D, :], precision=_HP)
    gpre += jnp.dot(glob, wg_ref[2 * D:, :], precision=_HP)
    gpre += bg_ref[...]
    go_ref[...] = glob + mult_ref[0, 0] * jnp.maximum(gpre, 0.0)


def _glob_tc(multa, nagg, eagg, glob, W_g, b_g2):
    return pl.pallas_call(
        _glob_body,
        out_shape=jax.ShapeDtypeStruct((G, D), _F32),
    )(multa, nagg, eagg, glob, W_g, b_g2)


def kernel(nodes, edges, receivers, senders, global_latent, node_graph_idx,
           edge_graph_idx, W_e, b_e, W_n, b_n, W_g, b_g, rn_w):
    nodes2 = nodes[0]
    edges2 = edges[0]
    recv = receivers[0]
    snd = senders[0]
    glob = global_latent[0]
    ng = node_graph_idx[0]
    eg = edge_graph_idx[0]
    multa = jax.nn.softplus(rn_w).reshape(1, 1)
    snd3 = snd.reshape(NW, NCHUNK, CH)
    rcv3 = recv.reshape(NW, NCHUNK, CH)

    ps, pr, gpe = _prep_tc(nodes2, glob, W_e, b_e.reshape(1, D))
    gs, gr = _make_gather_sc()(ps, pr, snd3, rcv3)
    eu, eo, eagg = _edge_tc(multa, edges2, gs, gr,
                            eg.reshape(NSTEP, 1, BLK), W_e[:D], gpe)
    adj = _make_scatter_sc()(eu, rcv3, jnp.zeros((N, D), _F32))
    no, nagg = _node_tc(multa, nodes2, adj, ng.reshape(NNSTEP, 1, NBLK), glob,
                        W_n, b_n.reshape(1, D))
    go = _glob_tc(multa, nagg, eagg, glob, W_g, b_g.reshape(1, D))
    return (no[None], eo[None], go[None])


# speedup vs baseline: 13.4960x; 1.1329x over previous
import functools

import jax
import jax.numpy as jnp
from jax import lax
from jax.experimental import pallas as pl
from jax.experimental.pallas import tpu as pltpu
from jax.experimental.pallas import tpu_sc as plsc

N = 10000
E = 320000
G = 16
D = 128

NH = 2
EH = E // NH

NC = 2
NS = 16
NW = NC * NS
ROWS_W = EH // NW
CH = 40
NCHUNK = ROWS_W // CH

RB = 5
NJ = NCHUNK // RB
KS = 2

BLK = 2000
NSTEP = EH // BLK

_HP = lax.Precision.HIGHEST
_F32 = jnp.float32


def _sc_mesh():
    return plsc.VectorSubcoreMesh(core_axis_name="c", subcore_axis_name="s")


def _prep_body(nodes_ref, glob_ref, we_ref, be_ref, ps_ref, pr_ref, gpe_ref):
    nodes = nodes_ref[...]
    ps_ref[...] = jnp.dot(nodes, we_ref[D:2 * D, :], precision=_HP)
    pr_ref[...] = jnp.dot(nodes, we_ref[2 * D:3 * D, :], precision=_HP)
    gpe_ref[...] = jnp.dot(glob_ref[...], we_ref[3 * D:, :], precision=_HP) + be_ref[...]


def _prep_tc(nodes, glob, W_e, b_e2):
    return pl.pallas_call(
        _prep_body,
        out_shape=(
            jax.ShapeDtypeStruct((N, D), _F32),
            jax.ShapeDtypeStruct((N, D), _F32),
            jax.ShapeDtypeStruct((G, D), _F32),
        ),
    )(nodes, glob, W_e, b_e2)


@functools.cache
def _make_gather_sc():
    @functools.partial(
        pl.kernel,
        out_type=(
            jax.ShapeDtypeStruct((EH, D), _F32),
            jax.ShapeDtypeStruct((EH, D), _F32),
        ),
        mesh=_sc_mesh(),
        scratch_types=[
            pltpu.VMEM((NCHUNK, CH), jnp.int32),
            pltpu.VMEM((NCHUNK, CH), jnp.int32),
            pltpu.VMEM((RB, CH, D), _F32),
            pltpu.VMEM((RB, CH, D), _F32),
            pltpu.SemaphoreType.DMA((RB,)),
            pltpu.SemaphoreType.DMA((RB,)),
            pltpu.SemaphoreType.DMA((RB,)),
            pltpu.SemaphoreType.DMA((RB,)),
        ],
    )
    def _gather_sc(ps, pr, snd3, rcv3, gs, gr, sidx, ridx, sbuf, rbuf,
                   gsemS, gsemR, wsemS, wsemR):
        wid = lax.axis_index("s") * NC + lax.axis_index("c")
        base = wid * ROWS_W
        pltpu.sync_copy(snd3.at[wid], sidx)
        pltpu.sync_copy(rcv3.at[wid], ridx)

        def _fire(c, k):
            pltpu.async_copy(ps.at[sidx.at[c]], sbuf.at[k], gsemS.at[k])
            pltpu.async_copy(pr.at[ridx.at[c]], rbuf.at[k], gsemR.at[k])

        def _drain_wb(c, k):
            off = base + c * CH
            pltpu.make_async_copy(ps.at[sidx.at[c]], sbuf.at[k],
                                  gsemS.at[k]).wait()
            pltpu.make_async_copy(pr.at[ridx.at[c]], rbuf.at[k],
                                  gsemR.at[k]).wait()
            pltpu.async_copy(sbuf.at[k], gs.at[pl.ds(off, CH)], wsemS.at[k])
            pltpu.async_copy(rbuf.at[k], gr.at[pl.ds(off, CH)], wsemR.at[k])

        def _wait_wb(c, k):
            off = base + c * CH
            pltpu.make_async_copy(sbuf.at[k], gs.at[pl.ds(off, CH)],
                                  wsemS.at[k]).wait()
            pltpu.make_async_copy(rbuf.at[k], gr.at[pl.ds(off, CH)],
                                  wsemR.at[k]).wait()

        for k in range(RB):
            _fire(k, k)

        def ring(j, carry):
            c0 = j * RB
            for k in range(RB):
                _drain_wb(c0 + k, k)
            for k in range(RB):
                _wait_wb(c0 + k, k)
                _fire(c0 + RB + k, k)
            return carry

        lax.fori_loop(0, NJ - 1, ring, 0)
        c0 = (NJ - 1) * RB
        for k in range(RB):
            _drain_wb(c0 + k, k)
        for k in range(RB):
            _wait_wb(c0 + k, k)

    return _gather_sc


def _edge_body(mult_ref, edges_ref, gs_ref, gr_ref, eg_ref, w1_ref, gpe_ref,
               eu_ref, eo_ref, eagg_ref):
    step = pl.program_id(0)
    ohT = (lax.broadcasted_iota(jnp.int32, (G, BLK), 0)
           == eg_ref[0, :, :]).astype(_F32)
    edges = edges_ref[...]
    pre = jnp.dot(edges, w1_ref[...], precision=_HP)
    pre += gs_ref[...] + gr_ref[...]
    pre += lax.dot_general(ohT, gpe_ref[...], (((0,), (0,)), ((), ())),
                           precision=_HP)
    eu = jnp.maximum(pre, 0.0)
    eu_ref[...] = eu
    eo_ref[...] = edges + mult_ref[0, 0] * eu

    @pl.when(step == 0)
    def _():
        eagg_ref[...] = jnp.zeros_like(eagg_ref)

    eagg_ref[...] += lax.dot_general(ohT, eu, (((1,), (0,)), ((), ())),
                                     precision=_HP)


def _edge_tc(half, multa, edges, gs, gr, eg3, W1, gpe, eo_prev):
    off = half * NSTEP
    in_specs = [
        pl.BlockSpec((1, 1), lambda i: (0, 0)),
        pl.BlockSpec((BLK, D), lambda i: (i + off, 0)),
        pl.BlockSpec((BLK, D), lambda i: (i, 0)),
        pl.BlockSpec((BLK, D), lambda i: (i, 0)),
        pl.BlockSpec((1, 1, BLK), lambda i: (i + off, 0, 0)),
        pl.BlockSpec((D, D), lambda i: (0, 0)),
        pl.BlockSpec((G, D), lambda i: (0, 0)),
    ]
    args = [multa, edges, gs, gr, eg3, W1, gpe]
    kwargs = {}
    if half == 0:
        body = _edge_body
    else:
        def body(mult_ref, edges_ref, gs_ref, gr_ref, eg_ref, w1_ref,
                 gpe_ref, eo_in_ref, eu_ref, eo_ref, eagg_ref):
            _edge_body(mult_ref, edges_ref, gs_ref, gr_ref, eg_ref, w1_ref,
                       gpe_ref, eu_ref, eo_ref, eagg_ref)

        in_specs.append(pl.BlockSpec(memory_space=pl.ANY))
        args.append(eo_prev)
        kwargs["input_output_aliases"] = {7: 1}
    return pl.pallas_call(
        body,
        grid=(NSTEP,),
        in_specs=in_specs,
        out_specs=[
            pl.BlockSpec((BLK, D), lambda i: (i, 0)),
            pl.BlockSpec((BLK, D), lambda i: (i + off, 0)),
            pl.BlockSpec((G, D), lambda i: (0, 0)),
        ],
        out_shape=(
            jax.ShapeDtypeStruct((EH, D), _F32),
            jax.ShapeDtypeStruct((E, D), _F32),
            jax.ShapeDtypeStruct((G, D), _F32),
        ),
        **kwargs,
    )(*args)


@functools.cache
def _make_scatter_sc():
    @functools.partial(
        pl.kernel,
        out_type=jax.ShapeDtypeStruct((NC, N, D), _F32),
        mesh=_sc_mesh(),
        scratch_types=[
            pltpu.VMEM((NCHUNK, CH), jnp.int32),
            pltpu.VMEM((KS, CH, D), _F32),
            pltpu.VMEM_SHARED((N, D), _F32),
            pltpu.SemaphoreType.DMA((KS,)),
        ],
    )
    def _scatter_sc(eu, rcv3, zer, adj, ridx, rows, accum, lsem):
        cid = lax.axis_index("c")
        sid = lax.axis_index("s")

        @pl.when(sid == 0)
        def _():
            pltpu.sync_copy(zer, accum)

        wid = sid * NC + cid
        base = wid * ROWS_W
        pltpu.sync_copy(rcv3.at[wid], ridx)
        plsc.subcore_barrier()

        def _fire(c, k):
            off = base + c * CH
            pltpu.async_copy(eu.at[pl.ds(off, CH)], rows.at[k], lsem.at[k])

        def _scat(c, k):
            off = base + c * CH
            pltpu.make_async_copy(eu.at[pl.ds(off, CH)], rows.at[k],
                                  lsem.at[k]).wait()
            pltpu.sync_copy(rows.at[k], accum.at[ridx.at[c]], add=True)

        for k in range(KS):
            _fire(k, k)

        nfull = NCHUNK // KS - 1

        def ring(j, carry):
            c0 = j * KS
            for k in range(KS):
                _scat(c0 + k, k)
                _fire(c0 + KS + k, k)
            return carry

        lax.fori_loop(0, nfull, ring, 0)
        for c in range(nfull * KS, NCHUNK):
            _scat(c, c % KS)
            nxt = c + KS
            if nxt < NCHUNK and nxt >= (nfull + 1) * KS:
                _fire(nxt, nxt % KS)
        plsc.subcore_barrier()

        @pl.when(sid == 0)
        def _():
            pltpu.sync_copy(accum, adj.at[cid])

    return _scatter_sc


NBLK = 2000
NNSTEP = N // NBLK


def _node_body(mult_ref, nodes_ref, adj0_ref, adj1_ref, ng_ref, glob_ref,
               wn_ref, bn_ref, no_ref, nagg_ref):
    step = pl.program_id(0)
    mult = mult_ref[0, 0]
    nodes = nodes_ref[...]
    adj = adj0_ref[0] + adj0_ref[1] + adj1_ref[0] + adj1_ref[1]
    ohT = (lax.broadcasted_iota(jnp.int32, (G, NBLK), 0)
           == ng_ref[0, :, :]).astype(_F32)
    gwn = jnp.dot(glob_ref[...], wn_ref[2 * D:, :], precision=_HP)
    pre = jnp.dot(nodes, wn_ref[:D, :], precision=_HP)
    pre += jnp.dot(adj, wn_ref[D:2 * D, :], precision=_HP)
    pre += lax.dot_general(ohT, gwn, (((0,), (0,)), ((), ())), precision=_HP)
    pre += bn_ref[...]
    nu = jnp.maximum(pre, 0.0)
    no_ref[...] = nodes + mult * nu

    @pl.when(step == 0)
    def _():
        nagg_ref[...] = jnp.zeros_like(nagg_ref)

    nagg_ref[...] += lax.dot_general(ohT, nu, (((1,), (0,)), ((), ())),
                                     precision=_HP)


def _node_tc(multa, nodes, adj0, adj1, ng3, glob, W_n, b_n2):
    return pl.pallas_call(
        _node_body,
        grid=(NNSTEP,),
        in_specs=[
            pl.BlockSpec((1, 1), lambda i: (0, 0)),
            pl.BlockSpec((NBLK, D), lambda i: (i, 0)),
            pl.BlockSpec((NC, NBLK, D), lambda i: (0, i, 0)),
            pl.BlockSpec((NC, NBLK, D), lambda i: (0, i, 0)),
            pl.BlockSpec((1, 1, NBLK), lambda i: (i, 0, 0)),
            pl.BlockSpec((G, D), lambda i: (0, 0)),
            pl.BlockSpec((3 * D, D), lambda i: (0, 0)),
            pl.BlockSpec((1, D), lambda i: (0, 0)),
        ],
        out_specs=[
            pl.BlockSpec((NBLK, D), lambda i: (i, 0)),
            pl.BlockSpec((G, D), lambda i: (0, 0)),
        ],
        out_shape=(
            jax.ShapeDtypeStruct((N, D), _F32),
            jax.ShapeDtypeStruct((G, D), _F32),
        ),
    )(multa, nodes, adj0, adj1, ng3, glob, W_n, b_n2)


def _glob_body(mult_ref, nagg_ref, ea0_ref, ea1_ref, glob_ref, wg_ref,
               bg_ref, go_ref):
    glob = glob_ref[...]
    gpre = jnp.dot(nagg_ref[...], wg_ref[:D, :], precision=_HP)
    gpre += jnp.dot(ea0_ref[...] + ea1_ref[...], wg_ref[D:2 * D, :],
                    precision=_HP)
    gpre += jnp.dot(glob, wg_ref[2 * D:, :], precision=_HP)
    gpre += bg_ref[...]
    go_ref[...] = glob + mult_ref[0, 0] * jnp.maximum(gpre, 0.0)


def _glob_tc(multa, nagg, ea0, ea1, glob, W_g, b_g2):
    return pl.pallas_call(
        _glob_body,
        out_shape=jax.ShapeDtypeStruct((G, D), _F32),
    )(multa, nagg, ea0, ea1, glob, W_g, b_g2)


def kernel(nodes, edges, receivers, senders, global_latent, node_graph_idx,
           edge_graph_idx, W_e, b_e, W_n, b_n, W_g, b_g, rn_w):
    nodes2 = nodes[0]
    edges2 = edges[0]
    glob = global_latent[0]
    ng = node_graph_idx[0]
    eg3 = edge_graph_idx[0].reshape(NH * NSTEP, 1, BLK)
    multa = jax.nn.softplus(rn_w).reshape(1, 1)
    snd4 = senders[0].reshape(NH, NW, NCHUNK, CH)
    rcv4 = receivers[0].reshape(NH, NW, NCHUNK, CH)
    zer = jnp.zeros((N, D), _F32)

    ps, pr, gpe = _prep_tc(nodes2, glob, W_e, b_e.reshape(1, D))
    gather = _make_gather_sc()
    scatter = _make_scatter_sc()

    gs0, gr0 = gather(ps, pr, snd4[0], rcv4[0])
    gs1, gr1 = gather(ps, pr, snd4[1], rcv4[1])
    eu0, eo0, ea0 = _edge_tc(0, multa, edges2, gs0, gr0, eg3, W_e[:D], gpe,
                             None)
    adj0 = scatter(eu0, rcv4[0], zer)
    eu1, eo, ea1 = _edge_tc(1, multa, edges2, gs1, gr1, eg3, W_e[:D], gpe,
                            eo0)
    adj1 = scatter(eu1, rcv4[1], zer)
    no, nagg = _node_tc(multa, nodes2, adj0, adj1,
                        ng.reshape(NNSTEP, 1, NBLK), glob, W_n,
                        b_n.reshape(1, D))
    go = _glob_tc(multa, nagg, ea0, ea1, glob, W_g, b_g.reshape(1, D))
    return (no[None], eo[None], go[None])


# speedup vs baseline: 14.6017x; 1.0819x over previous
import functools

import jax
import jax.numpy as jnp
from jax import lax
from jax.experimental import pallas as pl
from jax.experimental.pallas import tpu as pltpu
from jax.experimental.pallas import tpu_sc as plsc

N = 10000
E = 320000
G = 16
D = 128

NH = 5
EH = E // NH

NC = 2
NS = 16
NW = NC * NS
ROWS_W = EH // NW
CH = 40
NCHUNK = ROWS_W // CH

RB = 5
NJ = NCHUNK // RB
KS = 2

BLK = 2000
NSTEP = EH // BLK

_HP = lax.Precision.HIGHEST
_F32 = jnp.float32


def _sc_mesh():
    return plsc.VectorSubcoreMesh(core_axis_name="c", subcore_axis_name="s")


def _prep_body(nodes_ref, glob_ref, we_ref, be_ref, ps_ref, pr_ref, gpe_ref):
    nodes = nodes_ref[...]
    ps_ref[...] = jnp.dot(nodes, we_ref[D:2 * D, :], precision=_HP)
    pr_ref[...] = jnp.dot(nodes, we_ref[2 * D:3 * D, :], precision=_HP)
    gpe_ref[...] = jnp.dot(glob_ref[...], we_ref[3 * D:, :], precision=_HP) + be_ref[...]


def _prep_tc(nodes, glob, W_e, b_e2):
    return pl.pallas_call(
        _prep_body,
        out_shape=(
            jax.ShapeDtypeStruct((N, D), _F32),
            jax.ShapeDtypeStruct((N, D), _F32),
            jax.ShapeDtypeStruct((G, D), _F32),
        ),
    )(nodes, glob, W_e, b_e2)


@functools.cache
def _make_gather_sc():
    @functools.partial(
        pl.kernel,
        out_type=(
            jax.ShapeDtypeStruct((EH, D), _F32),
            jax.ShapeDtypeStruct((EH, D), _F32),
        ),
        mesh=_sc_mesh(),
        scratch_types=[
            pltpu.VMEM((NCHUNK, CH), jnp.int32),
            pltpu.VMEM((NCHUNK, CH), jnp.int32),
            pltpu.VMEM((RB, CH, D), _F32),
            pltpu.VMEM((RB, CH, D), _F32),
            pltpu.SemaphoreType.DMA((RB,)),
            pltpu.SemaphoreType.DMA((RB,)),
            pltpu.SemaphoreType.DMA((RB,)),
            pltpu.SemaphoreType.DMA((RB,)),
        ],
    )
    def _gather_sc(ps, pr, snd3, rcv3, gs, gr, sidx, ridx, sbuf, rbuf,
                   gsemS, gsemR, wsemS, wsemR):
        wid = lax.axis_index("s") * NC + lax.axis_index("c")
        base = wid * ROWS_W
        pltpu.sync_copy(snd3.at[wid], sidx)
        pltpu.sync_copy(rcv3.at[wid], ridx)

        def _fire(c, k):
            pltpu.async_copy(ps.at[sidx.at[c]], sbuf.at[k], gsemS.at[k])
            pltpu.async_copy(pr.at[ridx.at[c]], rbuf.at[k], gsemR.at[k])

        def _drain_wb(c, k):
            off = base + c * CH
            pltpu.make_async_copy(ps.at[sidx.at[c]], sbuf.at[k],
                                  gsemS.at[k]).wait()
            pltpu.make_async_copy(pr.at[ridx.at[c]], rbuf.at[k],
                                  gsemR.at[k]).wait()
            pltpu.async_copy(sbuf.at[k], gs.at[pl.ds(off, CH)], wsemS.at[k])
            pltpu.async_copy(rbuf.at[k], gr.at[pl.ds(off, CH)], wsemR.at[k])

        def _wait_wb(c, k):
            off = base + c * CH
            pltpu.make_async_copy(sbuf.at[k], gs.at[pl.ds(off, CH)],
                                  wsemS.at[k]).wait()
            pltpu.make_async_copy(rbuf.at[k], gr.at[pl.ds(off, CH)],
                                  wsemR.at[k]).wait()

        for k in range(RB):
            _fire(k, k)

        def ring(j, carry):
            c0 = j * RB
            for k in range(RB):
                _drain_wb(c0 + k, k)
            for k in range(RB):
                _wait_wb(c0 + k, k)
                _fire(c0 + RB + k, k)
            return carry

        lax.fori_loop(0, NJ - 1, ring, 0)
        c0 = (NJ - 1) * RB
        for k in range(RB):
            _drain_wb(c0 + k, k)
        for k in range(RB):
            _wait_wb(c0 + k, k)

    return _gather_sc


def _edge_body(mult_ref, edges_ref, gs_ref, gr_ref, eg_ref, w1_ref, gpe_ref,
               eu_ref, eo_ref, eagg_ref):
    step = pl.program_id(0)
    ohT = (lax.broadcasted_iota(jnp.int32, (G, BLK), 0)
           == eg_ref[0, :, :]).astype(_F32)
    edges = edges_ref[...]
    pre = jnp.dot(edges, w1_ref[...], precision=_HP)
    pre += gs_ref[...] + gr_ref[...]
    pre += lax.dot_general(ohT, gpe_ref[...], (((0,), (0,)), ((), ())),
                           precision=_HP)
    eu = jnp.maximum(pre, 0.0)
    eu_ref[...] = eu
    eo_ref[...] = edges + mult_ref[0, 0] * eu

    @pl.when(step == 0)
    def _():
        eagg_ref[...] = jnp.zeros_like(eagg_ref)

    eagg_ref[...] += lax.dot_general(ohT, eu, (((1,), (0,)), ((), ())),
                                     precision=_HP)


def _edge_tc(half, multa, edges, gs, gr, eg3, W1, gpe, eo_prev):
    off = half * NSTEP
    in_specs = [
        pl.BlockSpec((1, 1), lambda i: (0, 0)),
        pl.BlockSpec((BLK, D), lambda i: (i + off, 0)),
        pl.BlockSpec((BLK, D), lambda i: (i, 0)),
        pl.BlockSpec((BLK, D), lambda i: (i, 0)),
        pl.BlockSpec((1, 1, BLK), lambda i: (i + off, 0, 0)),
        pl.BlockSpec((D, D), lambda i: (0, 0)),
        pl.BlockSpec((G, D), lambda i: (0, 0)),
    ]
    args = [multa, edges, gs, gr, eg3, W1, gpe]
    kwargs = {}
    if half == 0:
        body = _edge_body
    else:
        def body(mult_ref, edges_ref, gs_ref, gr_ref, eg_ref, w1_ref,
                 gpe_ref, eo_in_ref, eu_ref, eo_ref, eagg_ref):
            _edge_body(mult_ref, edges_ref, gs_ref, gr_ref, eg_ref, w1_ref,
                       gpe_ref, eu_ref, eo_ref, eagg_ref)

        in_specs.append(pl.BlockSpec(memory_space=pl.ANY))
        args.append(eo_prev)
        kwargs["input_output_aliases"] = {7: 1}
    return pl.pallas_call(
        body,
        grid=(NSTEP,),
        in_specs=in_specs,
        out_specs=[
            pl.BlockSpec((BLK, D), lambda i: (i, 0)),
            pl.BlockSpec((BLK, D), lambda i: (i + off, 0)),
            pl.BlockSpec((G, D), lambda i: (0, 0)),
        ],
        out_shape=(
            jax.ShapeDtypeStruct((EH, D), _F32),
            jax.ShapeDtypeStruct((E, D), _F32),
            jax.ShapeDtypeStruct((G, D), _F32),
        ),
        **kwargs,
    )(*args)


@functools.cache
def _make_scatter_sc():
    @functools.partial(
        pl.kernel,
        out_type=jax.ShapeDtypeStruct((NC, N, D), _F32),
        mesh=_sc_mesh(),
        scratch_types=[
            pltpu.VMEM((NCHUNK, CH), jnp.int32),
            pltpu.VMEM((KS, CH, D), _F32),
            pltpu.VMEM_SHARED((N, D), _F32),
            pltpu.SemaphoreType.DMA((KS,)),
        ],
    )
    def _scatter_sc(eu, rcv3, zer, adj, ridx, rows, accum, lsem):
        cid = lax.axis_index("c")
        sid = lax.axis_index("s")

        @pl.when(sid == 0)
        def _():
            pltpu.sync_copy(zer, accum)

        wid = sid * NC + cid
        base = wid * ROWS_W
        pltpu.sync_copy(rcv3.at[wid], ridx)
        plsc.subcore_barrier()

        def _fire(c, k):
            off = base + c * CH
            pltpu.async_copy(eu.at[pl.ds(off, CH)], rows.at[k], lsem.at[k])

        def _scat(c, k):
            off = base + c * CH
            pltpu.make_async_copy(eu.at[pl.ds(off, CH)], rows.at[k],
                                  lsem.at[k]).wait()
            pltpu.sync_copy(rows.at[k], accum.at[ridx.at[c]], add=True)

        for k in range(KS):
            _fire(k, k)

        nfull = NCHUNK // KS - 1

        def ring(j, carry):
            c0 = j * KS
            for k in range(KS):
                _scat(c0 + k, k)
                _fire(c0 + KS + k, k)
            return carry

        lax.fori_loop(0, nfull, ring, 0)
        for c in range(nfull * KS, NCHUNK):
            _scat(c, c % KS)
            nxt = c + KS
            if nxt < NCHUNK and nxt >= (nfull + 1) * KS:
                _fire(nxt, nxt % KS)
        plsc.subcore_barrier()

        @pl.when(sid == 0)
        def _():
            pltpu.sync_copy(accum, adj.at[cid])

    return _scatter_sc


NBLK = 2000
NNSTEP = N // NBLK


def _node_body(*refs):
    (mult_ref, nodes_ref), adj_refs = refs[:2], refs[2:2 + NH]
    (ng_ref, glob_ref, wn_ref, bn_ref, no_ref, nagg_ref) = refs[2 + NH:]
    step = pl.program_id(0)
    mult = mult_ref[0, 0]
    nodes = nodes_ref[...]
    adj = adj_refs[0][0] + adj_refs[0][1]
    for a in adj_refs[1:]:
        adj += a[0] + a[1]
    ohT = (lax.broadcasted_iota(jnp.int32, (G, NBLK), 0)
           == ng_ref[0, :, :]).astype(_F32)
    gwn = jnp.dot(glob_ref[...], wn_ref[2 * D:, :], precision=_HP)
    pre = jnp.dot(nodes, wn_ref[:D, :], precision=_HP)
    pre += jnp.dot(adj, wn_ref[D:2 * D, :], precision=_HP)
    pre += lax.dot_general(ohT, gwn, (((0,), (0,)), ((), ())), precision=_HP)
    pre += bn_ref[...]
    nu = jnp.maximum(pre, 0.0)
    no_ref[...] = nodes + mult * nu

    @pl.when(step == 0)
    def _():
        nagg_ref[...] = jnp.zeros_like(nagg_ref)

    nagg_ref[...] += lax.dot_general(ohT, nu, (((1,), (0,)), ((), ())),
                                     precision=_HP)


def _node_tc(multa, nodes, adjs, ng3, glob, W_n, b_n2):
    return pl.pallas_call(
        _node_body,
        grid=(NNSTEP,),
        in_specs=[
            pl.BlockSpec((1, 1), lambda i: (0, 0)),
            pl.BlockSpec((NBLK, D), lambda i: (i, 0)),
        ] + [
            pl.BlockSpec((NC, NBLK, D), lambda i: (0, i, 0))
            for _ in range(NH)
        ] + [
            pl.BlockSpec((1, 1, NBLK), lambda i: (i, 0, 0)),
            pl.BlockSpec((G, D), lambda i: (0, 0)),
            pl.BlockSpec((3 * D, D), lambda i: (0, 0)),
            pl.BlockSpec((1, D), lambda i: (0, 0)),
        ],
        out_specs=[
            pl.BlockSpec((NBLK, D), lambda i: (i, 0)),
            pl.BlockSpec((G, D), lambda i: (0, 0)),
        ],
        out_shape=(
            jax.ShapeDtypeStruct((N, D), _F32),
            jax.ShapeDtypeStruct((G, D), _F32),
        ),
    )(multa, nodes, *adjs, ng3, glob, W_n, b_n2)


def _glob_body(*refs):
    (mult_ref, nagg_ref), ea_refs = refs[:2], refs[2:2 + NH]
    (glob_ref, wg_ref, bg_ref, go_ref) = refs[2 + NH:]
    glob = glob_ref[...]
    ea = ea_refs[0][...]
    for e in ea_refs[1:]:
        ea += e[...]
    gpre = jnp.dot(nagg_ref[...], wg_ref[:D, :], precision=_HP)
    gpre += jnp.dot(ea, wg_ref[D:2 * D, :], precision=_HP)
    gpre += jnp.dot(glob, wg_ref[2 * D:, :], precision=_HP)
    gpre += bg_ref[...]
    go_ref[...] = glob + mult_ref[0, 0] * jnp.maximum(gpre, 0.0)


def _glob_tc(multa, nagg, eas, glob, W_g, b_g2):
    return pl.pallas_call(
        _glob_body,
        out_shape=jax.ShapeDtypeStruct((G, D), _F32),
    )(multa, nagg, *eas, glob, W_g, b_g2)


def kernel(nodes, edges, receivers, senders, global_latent, node_graph_idx,
           edge_graph_idx, W_e, b_e, W_n, b_n, W_g, b_g, rn_w):
    nodes2 = nodes[0]
    edges2 = edges[0]
    glob = global_latent[0]
    ng = node_graph_idx[0]
    eg3 = edge_graph_idx[0].reshape(NH * NSTEP, 1, BLK)
    multa = jax.nn.softplus(rn_w).reshape(1, 1)
    snd4 = senders[0].reshape(NH, NW, NCHUNK, CH)
    rcv4 = receivers[0].reshape(NH, NW, NCHUNK, CH)
    zer = jnp.zeros((N, D), _F32)

    ps, pr, gpe = _prep_tc(nodes2, glob, W_e, b_e.reshape(1, D))
    gather = _make_gather_sc()
    scatter = _make_scatter_sc()

    gsr = [gather(ps, pr, snd4[h], rcv4[h]) for h in range(NH)]
    adjs, eas = [], []
    eo = None
    for h in range(NH):
        gs, gr = gsr[h]
        eu, eo, ea = _edge_tc(h, multa, edges2, gs, gr, eg3, W_e[:D], gpe,
                              eo)
        adjs.append(scatter(eu, rcv4[h], zer))
        eas.append(ea)
    no, nagg = _node_tc(multa, nodes2, adjs,
                        ng.reshape(NNSTEP, 1, NBLK), glob, W_n,
                        b_n.reshape(1, D))
    go = _glob_tc(multa, nagg, eas, glob, W_g, b_g.reshape(1, D))
    return (no[None], eo[None], go[None])
